# baseline (device time: 77348 ns/iter reference)
import jax
import jax.numpy as jnp
from jax import lax
from jax.experimental import pallas as pl
from jax.experimental.pallas import tpu as pltpu

N_DEV = 32
SQ = 1024
DM = 1024
H_LOC = 8
DH = 128
CHUNK = SQ // N_DEV
WINDOW = 128
SCALE = 0.08838834764831843

QB = 4
QBS = SQ // QB
KW = QBS + 2 * WINDOW
CPB = QBS // CHUNK

_CompilerParams = getattr(pltpu, "CompilerParams", None) or getattr(
    pltpu, "TPUCompilerParams"
)

_MESH = pl.DeviceIdType.MESH


def kernel(x, Wq, K_ext, V_ext, Wo):
    def body(x_ref, wq_ref, kext_ref, vext_ref, wo_ref, out_ref,
             part_ref, ctx_ref, myred_ref, qs_ref, kbf_ref, vbf_ref,
             kf32_ref, vf32_ref, wobf_ref, red_ref, rs_buf, ag_buf,
             kv_sems, rs_send, rs_recv, ag_send, ag_recv):
        me = lax.axis_index("i")

        barrier = pltpu.get_barrier_semaphore()
        for o in range(1, N_DEV):
            pl.semaphore_signal(barrier, inc=1,
                                device_id=(lax.rem(me + o, N_DEV),),
                                device_id_type=_MESH)

        kv_dmas = {}
        for h in range(H_LOC):
            hidx = me * H_LOC + h
            dk = pltpu.make_async_copy(
                kext_ref.at[:, hidx, :], kf32_ref.at[h], kv_sems.at[h])
            dv = pltpu.make_async_copy(
                vext_ref.at[:, hidx, :], vf32_ref.at[h],
                kv_sems.at[H_LOC + h])
            dk.start()
            dv.start()
            kv_dmas[h] = (dk, dv)

        q = jnp.dot(x_ref[...].astype(jnp.bfloat16),
                    wq_ref[...].astype(jnp.bfloat16),
                    preferred_element_type=jnp.float32)
        qs_ref[...] = (q * SCALE).astype(jnp.bfloat16)
        wobf_ref[...] = wo_ref[...].astype(jnp.bfloat16)

        grp = lax.div(me, CPB)
        for b in range(QB):
            qb = lax.rem(grp + b, QB)
            r0 = pl.multiple_of(qb * QBS, QBS)
            kstart = pl.multiple_of(jnp.clip(r0 - WINDOW, 0, SQ - KW), WINDOW)
            qi = r0 + lax.broadcasted_iota(jnp.int32, (QBS, KW), 0)
            kj = kstart + lax.broadcasted_iota(jnp.int32, (QBS, KW), 1)
            mask = jnp.abs(qi - kj) <= WINDOW
            for h in range(H_LOC):
                if b == 0:
                    dk, dv = kv_dmas[h]
                    dk.wait()
                    dv.wait()
                    kbf_ref[h] = kf32_ref[h].astype(jnp.bfloat16)
                    vbf_ref[h] = vf32_ref[h].astype(jnp.bfloat16)
                qh = qs_ref[pl.ds(r0, QBS), h * DH:(h + 1) * DH]
                s = lax.dot_general(
                    qh, kbf_ref[h, pl.ds(kstart, KW), :],
                    (((1,), (1,)), ((), ())),
                    preferred_element_type=jnp.float32)
                w = jnp.exp(jnp.where(mask, s, -1e9))
                inv = 1.0 / jnp.sum(w, axis=1, keepdims=True)
                ctx_ref[:, h * DH:(h + 1) * DH] = (
                    jnp.dot(w.astype(jnp.bfloat16),
                            vbf_ref[h, pl.ds(kstart, KW), :],
                            preferred_element_type=jnp.float32)
                    * inv).astype(jnp.bfloat16)
            pblk = jnp.dot(ctx_ref[...], wobf_ref[...],
                           preferred_element_type=jnp.float32)
            part_ref[pl.ds(qb * CPB, CPB)] = (
                pblk.astype(jnp.bfloat16).reshape(CPB, CHUNK, DM))

            if b == 0:
                pl.semaphore_wait(barrier, N_DEV - 1)

            for t in range(CPB):
                c = qb * CPB + t
                slot = lax.rem(me + (2 * N_DEV - 1) - c, N_DEV)

                @pl.when(c != me)
                def _(c=c, slot=slot):
                    rdma = pltpu.make_async_remote_copy(
                        src_ref=part_ref.at[c],
                        dst_ref=rs_buf.at[slot],
                        send_sem=rs_send.at[c],
                        recv_sem=rs_recv.at[slot],
                        device_id=(c,),
                        device_id_type=_MESH,
                    )
                    rdma.start()

            if b == 0:
                red_ref[...] = part_ref[me].astype(jnp.float32)

            for w in range(max(b - 1, 0), b):
                g_w = lax.rem(grp + QB - w, QB)
                for t in range(CPB):
                    j = g_w * CPB + t
                    slot = lax.rem(j + 2 * N_DEV - 1 - me, N_DEV)

                    @pl.when(j != me)
                    def _(j=j, slot=slot):
                        recv = pltpu.make_async_remote_copy(
                            src_ref=rs_buf.at[slot], dst_ref=rs_buf.at[slot],
                            send_sem=rs_send.at[0], recv_sem=rs_recv.at[slot],
                            device_id=(me,), device_id_type=_MESH,
                        )
                        recv.wait_recv()
                        red_ref[...] = red_ref[...] + rs_buf[slot].astype(
                            jnp.float32)

        for w in range(QB - 1, QB):
            g_w = lax.rem(grp + QB - w, QB)
            for t in range(CPB):
                j = g_w * CPB + t
                slot = lax.rem(j + 2 * N_DEV - 1 - me, N_DEV)

                @pl.when(j != me)
                def _(j=j, slot=slot):
                    recv = pltpu.make_async_remote_copy(
                        src_ref=rs_buf.at[slot], dst_ref=rs_buf.at[slot],
                        send_sem=rs_send.at[0], recv_sem=rs_recv.at[slot],
                        device_id=(me,), device_id_type=_MESH,
                    )
                    recv.wait_recv()
                    red_ref[...] = red_ref[...] + rs_buf[slot].astype(
                        jnp.float32)

        red = red_ref[...]
        myred_ref[...] = red.astype(jnp.bfloat16)

        for o in range(1, N_DEV):
            d = lax.rem(me + o, N_DEV)
            slot = N_DEV - 1 - o
            rdma = pltpu.make_async_remote_copy(
                src_ref=myred_ref,
                dst_ref=ag_buf.at[slot],
                send_sem=ag_send.at[o - 1],
                recv_sem=ag_recv.at[slot],
                device_id=(d,),
                device_id_type=_MESH,
            )
            rdma.start()

        for c in range(N_DEV):
            @pl.when(c != me)
            def _(c=c):
                snd = pltpu.make_async_remote_copy(
                    src_ref=part_ref.at[c], dst_ref=rs_buf.at[0],
                    send_sem=rs_send.at[c], recv_sem=rs_recv.at[0],
                    device_id=(me,), device_id_type=_MESH,
                )
                snd.wait_send()

        out_ref[me] = red

        for s in range(N_DEV - 1):
            recv = pltpu.make_async_remote_copy(
                src_ref=myred_ref, dst_ref=ag_buf.at[s],
                send_sem=ag_send.at[s], recv_sem=ag_recv.at[s],
                device_id=(me,), device_id_type=_MESH,
            )
            recv.wait_recv()
            c = lax.rem(me + s + 1, N_DEV)
            out_ref[c] = ag_buf[s].astype(jnp.float32)

        for s in range(N_DEV - 1):
            snd = pltpu.make_async_remote_copy(
                src_ref=myred_ref, dst_ref=ag_buf.at[s],
                send_sem=ag_send.at[s], recv_sem=ag_recv.at[s],
                device_id=(me,), device_id_type=_MESH,
            )
            snd.wait_send()

    out = pl.pallas_call(
        body,
        out_shape=jax.ShapeDtypeStruct((N_DEV, CHUNK, DM), jnp.float32),
        in_specs=[
            pl.BlockSpec(memory_space=pltpu.VMEM),
            pl.BlockSpec(memory_space=pltpu.VMEM),
            pl.BlockSpec(memory_space=pl.ANY),
            pl.BlockSpec(memory_space=pl.ANY),
            pl.BlockSpec(memory_space=pltpu.VMEM),
        ],
        out_specs=pl.BlockSpec(memory_space=pltpu.VMEM),
        scratch_shapes=[
            pltpu.VMEM((N_DEV, CHUNK, DM), jnp.bfloat16),
            pltpu.VMEM((QBS, H_LOC * DH), jnp.bfloat16),
            pltpu.VMEM((CHUNK, DM), jnp.bfloat16),
            pltpu.VMEM((SQ, DM), jnp.bfloat16),
            pltpu.VMEM((H_LOC, SQ, DH), jnp.bfloat16),
            pltpu.VMEM((H_LOC, SQ, DH), jnp.bfloat16),
            pltpu.VMEM((H_LOC, SQ, DH), jnp.float32),
            pltpu.VMEM((H_LOC, SQ, DH), jnp.float32),
            pltpu.VMEM((DM, DM), jnp.bfloat16),
            pltpu.VMEM((CHUNK, DM), jnp.float32),
            pltpu.VMEM((N_DEV - 1, CHUNK, DM), jnp.bfloat16),
            pltpu.VMEM((N_DEV - 1, CHUNK, DM), jnp.bfloat16),
            pltpu.SemaphoreType.DMA((2 * H_LOC,)),
            pltpu.SemaphoreType.DMA((N_DEV,)),
            pltpu.SemaphoreType.DMA((N_DEV - 1,)),
            pltpu.SemaphoreType.DMA((N_DEV - 1,)),
            pltpu.SemaphoreType.DMA((N_DEV - 1,)),
        ],
        compiler_params=_CompilerParams(collective_id=0),
    )(x[0], Wq, K_ext[0], V_ext[0], Wo)
    return out.reshape(1, SQ, DM)


# device time: 75188 ns/iter; 1.0287x vs baseline; 1.0287x over previous
import jax
import jax.numpy as jnp
from jax import lax
from jax.experimental import pallas as pl
from jax.experimental.pallas import tpu as pltpu

N_DEV = 32
SQ = 1024
DM = 1024
H_LOC = 8
DH = 128
CHUNK = SQ // N_DEV
WINDOW = 128
SCALE = 0.08838834764831843

QB = 4
QBS = SQ // QB
KW = QBS + 2 * WINDOW
CPB = QBS // CHUNK

_CompilerParams = getattr(pltpu, "CompilerParams", None) or getattr(
    pltpu, "TPUCompilerParams"
)

_MESH = pl.DeviceIdType.MESH


def kernel(x, Wq, K_ext, V_ext, Wo):
    def body(x_ref, wq_ref, kext_ref, vext_ref, wo_ref, out_ref,
             part_ref, ctx_ref, myred_ref, qs_ref, kbf_ref, vbf_ref,
             kf32_ref, vf32_ref, wobf_ref, rs_buf, ag_buf,
             kv_sems, rs_send, rs_recv, ag_send, ag_recv):
        me = lax.axis_index("i")

        barrier = pltpu.get_barrier_semaphore()
        for o in range(1, N_DEV):
            pl.semaphore_signal(barrier, inc=1,
                                device_id=(lax.rem(me + o, N_DEV),),
                                device_id_type=_MESH)

        kv_dmas = {}
        for h in range(H_LOC):
            hidx = me * H_LOC + h
            dk = pltpu.make_async_copy(
                kext_ref.at[:, hidx, :], kf32_ref.at[h], kv_sems.at[h])
            dv = pltpu.make_async_copy(
                vext_ref.at[:, hidx, :], vf32_ref.at[h],
                kv_sems.at[H_LOC + h])
            dk.start()
            dv.start()
            kv_dmas[h] = (dk, dv)

        q = jnp.dot(x_ref[...].astype(jnp.bfloat16),
                    wq_ref[...].astype(jnp.bfloat16),
                    preferred_element_type=jnp.float32)
        qs_ref[...] = (q * SCALE).astype(jnp.bfloat16)
        wobf_ref[...] = wo_ref[...].astype(jnp.bfloat16)

        grp = lax.div(me, CPB)
        for b in range(QB):
            qb = lax.rem(grp + b, QB)
            r0 = pl.multiple_of(qb * QBS, QBS)
            kstart = pl.multiple_of(jnp.clip(r0 - WINDOW, 0, SQ - KW), WINDOW)
            qi = r0 + lax.broadcasted_iota(jnp.int32, (QBS, KW), 0)
            kj = kstart + lax.broadcasted_iota(jnp.int32, (QBS, KW), 1)
            mask = jnp.abs(qi - kj) <= WINDOW
            for h in range(H_LOC):
                if b == 0:
                    dk, dv = kv_dmas[h]
                    dk.wait()
                    dv.wait()
                    kbf_ref[h] = kf32_ref[h].astype(jnp.bfloat16)
                    vbf_ref[h] = vf32_ref[h].astype(jnp.bfloat16)
                qh = qs_ref[pl.ds(r0, QBS), h * DH:(h + 1) * DH]
                s = lax.dot_general(
                    qh, kbf_ref[h, pl.ds(kstart, KW), :],
                    (((1,), (1,)), ((), ())),
                    preferred_element_type=jnp.float32)
                w = jnp.exp(jnp.where(mask, s, -1e9))
                inv = 1.0 / jnp.sum(w, axis=1, keepdims=True)
                ctx_ref[:, h * DH:(h + 1) * DH] = (
                    jnp.dot(w.astype(jnp.bfloat16),
                            vbf_ref[h, pl.ds(kstart, KW), :],
                            preferred_element_type=jnp.float32)
                    * inv).astype(jnp.bfloat16)
            pblk = jnp.dot(ctx_ref[...], wobf_ref[...],
                           preferred_element_type=jnp.float32)
            part_ref[pl.ds(qb * CPB, CPB)] = (
                pblk.astype(jnp.bfloat16).reshape(CPB, CHUNK, DM))

            if b == 0:
                pl.semaphore_wait(barrier, N_DEV - 1)

            for t in range(CPB):
                c = qb * CPB + t
                slot = lax.rem(me + (2 * N_DEV - 1) - c, N_DEV)

                @pl.when(c != me)
                def _(c=c, slot=slot):
                    rdma = pltpu.make_async_remote_copy(
                        src_ref=part_ref.at[c],
                        dst_ref=rs_buf.at[slot],
                        send_sem=rs_send.at[c],
                        recv_sem=rs_recv.at[slot],
                        device_id=(c,),
                        device_id_type=_MESH,
                    )
                    rdma.start()

        red = part_ref[me].astype(jnp.float32)
        for s in range(N_DEV - 1):
            recv = pltpu.make_async_remote_copy(
                src_ref=rs_buf.at[s], dst_ref=rs_buf.at[s],
                send_sem=rs_send.at[0], recv_sem=rs_recv.at[s],
                device_id=(me,), device_id_type=_MESH,
            )
            recv.wait_recv()
            red = red + rs_buf[s].astype(jnp.float32)
        myred_ref[...] = red.astype(jnp.bfloat16)

        for o in range(1, N_DEV):
            d = lax.rem(me + o, N_DEV)
            slot = N_DEV - 1 - o
            rdma = pltpu.make_async_remote_copy(
                src_ref=myred_ref,
                dst_ref=ag_buf.at[slot],
                send_sem=ag_send.at[o - 1],
                recv_sem=ag_recv.at[slot],
                device_id=(d,),
                device_id_type=_MESH,
            )
            rdma.start()

        for c in range(N_DEV):
            @pl.when(c != me)
            def _(c=c):
                snd = pltpu.make_async_remote_copy(
                    src_ref=part_ref.at[c], dst_ref=rs_buf.at[0],
                    send_sem=rs_send.at[c], recv_sem=rs_recv.at[0],
                    device_id=(me,), device_id_type=_MESH,
                )
                snd.wait_send()

        out_ref[me] = red

        for s in range(N_DEV - 1):
            recv = pltpu.make_async_remote_copy(
                src_ref=myred_ref, dst_ref=ag_buf.at[s],
                send_sem=ag_send.at[s], recv_sem=ag_recv.at[s],
                device_id=(me,), device_id_type=_MESH,
            )
            recv.wait_recv()
            c = lax.rem(me + s + 1, N_DEV)
            out_ref[c] = ag_buf[s].astype(jnp.float32)

        for s in range(N_DEV - 1):
            snd = pltpu.make_async_remote_copy(
                src_ref=myred_ref, dst_ref=ag_buf.at[s],
                send_sem=ag_send.at[s], recv_sem=ag_recv.at[s],
                device_id=(me,), device_id_type=_MESH,
            )
            snd.wait_send()

    out = pl.pallas_call(
        body,
        out_shape=jax.ShapeDtypeStruct((N_DEV, CHUNK, DM), jnp.float32),
        in_specs=[
            pl.BlockSpec(memory_space=pltpu.VMEM),
            pl.BlockSpec(memory_space=pltpu.VMEM),
            pl.BlockSpec(memory_space=pl.ANY),
            pl.BlockSpec(memory_space=pl.ANY),
            pl.BlockSpec(memory_space=pltpu.VMEM),
        ],
        out_specs=pl.BlockSpec(memory_space=pltpu.VMEM),
        scratch_shapes=[
            pltpu.VMEM((N_DEV, CHUNK, DM), jnp.bfloat16),
            pltpu.VMEM((QBS, H_LOC * DH), jnp.bfloat16),
            pltpu.VMEM((CHUNK, DM), jnp.bfloat16),
            pltpu.VMEM((SQ, DM), jnp.bfloat16),
            pltpu.VMEM((H_LOC, SQ, DH), jnp.bfloat16),
            pltpu.VMEM((H_LOC, SQ, DH), jnp.bfloat16),
            pltpu.VMEM((H_LOC, SQ, DH), jnp.float32),
            pltpu.VMEM((H_LOC, SQ, DH), jnp.float32),
            pltpu.VMEM((DM, DM), jnp.bfloat16),
            pltpu.VMEM((N_DEV - 1, CHUNK, DM), jnp.bfloat16),
            pltpu.VMEM((N_DEV - 1, CHUNK, DM), jnp.bfloat16),
            pltpu.SemaphoreType.DMA((2 * H_LOC,)),
            pltpu.SemaphoreType.DMA((N_DEV,)),
            pltpu.SemaphoreType.DMA((N_DEV - 1,)),
            pltpu.SemaphoreType.DMA((N_DEV - 1,)),
            pltpu.SemaphoreType.DMA((N_DEV - 1,)),
        ],
        compiler_params=_CompilerParams(collective_id=0),
    )(x[0], Wq, K_ext[0], V_ext[0], Wo)
    return out.reshape(1, SQ, DM)
